# baseline (speedup 1.0000x reference)
import functools

import jax
import jax.numpy as jnp
from jax import lax
from jax.experimental import pallas as pl
from jax.experimental.pallas import tpu as pltpu
from jax.experimental.pallas import tpu_sc as plsc

N = 50000
E = 800000
IN_DIM = 128
H = 108
OUT_DIM = 108

DP = 128
ONES_COL = 108
BR = 1024
NP = 50176
NBLK = NP // BR

NUM_CHUNKS = 8
CHUNK = NP // NUM_CHUNKS
PASSES = NUM_CHUNKS // 2
ACC_ROWS = CHUNK + 128
ZPT = 80
CPT = 56

FIRE = 128
STCAP = 272
EROWS = 6272
EP = EROWS * 128
TROWS = EROWS // 16
BLKR = 56
NBLKS = TROWS // BLKR

_f32 = jnp.float32


def _mm(a, b):
    return lax.dot(a, b, precision=lax.Precision.HIGHEST,
                   preferred_element_type=_f32)


def _ones_col(m):
    col = lax.broadcasted_iota(jnp.int32, m.shape, 1)
    return jnp.where(col == ONES_COL, 1.0, m)


def _embed_pool_body(x_ref, we_ref, be_ref, wp_ref, bp_ref, h_ref, m_ref):
    h = _mm(x_ref[...], we_ref[...]) + be_ref[...]
    h_ref[...] = h
    m = jnp.maximum(_mm(h, wp_ref[...]) + bp_ref[...], 0.0)
    m_ref[...] = _ones_col(m)


def _apply_block(h, s, wah, wac, ba):
    deg = s[:, ONES_COL:ONES_COL + 1]
    c = s * (1.0 / jnp.maximum(deg, 1.0))
    bundle = _mm(h, wah) + _mm(c, wac) + ba
    nrm = jnp.sqrt(jnp.sum(bundle * bundle, axis=1, keepdims=True))
    bundle = bundle / jnp.maximum(nrm, 1e-12)
    return h + jnp.maximum(bundle, 0.0)


def _apply_pool_body(h_ref, s_ref, wah_ref, wac_ref, ba_ref, wp_ref, bp_ref,
                     h1_ref, m1_ref):
    h1 = _apply_block(h_ref[...], s_ref[...], wah_ref[...], wac_ref[...],
                      ba_ref[...])
    h1_ref[...] = h1
    m = jnp.maximum(_mm(h1, wp_ref[...]) + bp_ref[...], 0.0)
    m1_ref[...] = _ones_col(m)


def _final_body(h_ref, s_ref, wah_ref, wac_ref, ba_ref, out_ref):
    pi = pl.program_id(0)
    h2 = _apply_block(h_ref[...], s_ref[...], wah_ref[...], wac_ref[...],
                      ba_ref[...])
    row = pi * BR + lax.broadcasted_iota(jnp.int32, h2.shape, 0)
    h2 = jnp.where(row < N, h2, 0.0)
    part = jnp.sum(h2, axis=0, keepdims=True)

    @pl.when(pi == 0)
    def _():
        out_ref[...] = jnp.zeros_like(out_ref)

    acc = out_ref[...] + part

    @pl.when(pi == NBLK - 1)
    def _():
        out_ref[...] = acc * (1.0 / N)

    @pl.when(pi < NBLK - 1)
    def _():
        out_ref[...] = acc


def _row_spec(i):
    return (i, 0)


def _fixed_spec(i):
    return (0, 0)


def _embed_pool(x, we, be, wp, bp):
    return pl.pallas_call(
        _embed_pool_body,
        grid=(NBLK,),
        in_specs=[
            pl.BlockSpec((BR, IN_DIM), _row_spec),
            pl.BlockSpec((IN_DIM, DP), _fixed_spec),
            pl.BlockSpec((1, DP), _fixed_spec),
            pl.BlockSpec((DP, DP), _fixed_spec),
            pl.BlockSpec((1, DP), _fixed_spec),
        ],
        out_specs=[pl.BlockSpec((BR, DP), _row_spec),
                   pl.BlockSpec((BR, DP), _row_spec)],
        out_shape=[jax.ShapeDtypeStruct((NP, DP), _f32)] * 2,
    )(x, we, be, wp, bp)


def _apply_pool(h, s, wah, wac, ba, wp, bp):
    return pl.pallas_call(
        _apply_pool_body,
        grid=(NBLK,),
        in_specs=[
            pl.BlockSpec((BR, DP), _row_spec),
            pl.BlockSpec((BR, DP), _row_spec),
            pl.BlockSpec((DP, DP), _fixed_spec),
            pl.BlockSpec((DP, DP), _fixed_spec),
            pl.BlockSpec((1, DP), _fixed_spec),
            pl.BlockSpec((DP, DP), _fixed_spec),
            pl.BlockSpec((1, DP), _fixed_spec),
        ],
        out_specs=[pl.BlockSpec((BR, DP), _row_spec),
                   pl.BlockSpec((BR, DP), _row_spec)],
        out_shape=[jax.ShapeDtypeStruct((NP, DP), _f32)] * 2,
    )(h, s, wah, wac, ba, wp, bp)


def _final(h, s, wah, wac, ba):
    return pl.pallas_call(
        _final_body,
        grid=(NBLK,),
        in_specs=[
            pl.BlockSpec((BR, DP), _row_spec),
            pl.BlockSpec((BR, DP), _row_spec),
            pl.BlockSpec((DP, DP), _fixed_spec),
            pl.BlockSpec((DP, DP), _fixed_spec),
            pl.BlockSpec((1, DP), _fixed_spec),
        ],
        out_specs=pl.BlockSpec((1, DP), _fixed_spec),
        out_shape=jax.ShapeDtypeStruct((1, DP), _f32),
    )(h, s, wah, wac, ba)


_SC_MESH = plsc.VectorSubcoreMesh(core_axis_name="c", subcore_axis_name="s")


@functools.partial(
    pl.kernel,
    out_type=jax.ShapeDtypeStruct((NP, DP), _f32),
    mesh=_SC_MESH,
    scratch_types=[
        pltpu.VMEM_SHARED((ACC_ROWS, DP), _f32),
        pltpu.VMEM((2, FIRE, DP), _f32),
        pltpu.VMEM((2 * BLKR, 128), jnp.int32),
        pltpu.VMEM((2 * BLKR, 128), jnp.int32),
        pltpu.VMEM((STCAP,), jnp.int32),
        pltpu.VMEM((STCAP,), jnp.int32),
        pltpu.VMEM((2, FIRE), jnp.int32),
        pltpu.VMEM((2, FIRE), jnp.int32),
        pltpu.SemaphoreType.DMA,
        pltpu.SemaphoreType.DMA,
        pltpu.SemaphoreType.DMA,
    ],
    compiler_params=pltpu.CompilerParams(needs_layout_passes=False),
)
def _edge_pass(src_hbm, dst_hbm, m_hbm, out_hbm,
               acc, rows_v, bsrc, bdst, st_src, st_dst, f_src, f_dst,
               sem_s, sem_g, sem_i):
    cid = lax.axis_index("c")
    sid = lax.axis_index("s")

    def scatter_wait():
        pltpu.make_async_copy(rows_v.at[0], acc.at[f_dst.at[0]], sem_s).wait()

    def gather_wait():
        pltpu.make_async_copy(m_hbm.at[f_src.at[0]], rows_v.at[0],
                              sem_g).wait()

    def fire(b, pad_cnt):
        scatter_wait()
        for j in range(FIRE // 16):
            sv = st_src[pl.ds(j * 16, 16)]
            dv = st_dst[pl.ds(j * 16, 16)]
            if pad_cnt is not None:
                i16 = lax.broadcasted_iota(jnp.int32, (16,), 0)
                keep = (j * 16 + i16) < pad_cnt
                sv = jnp.where(keep, sv, 0)
                dv = jnp.where(keep, dv, CHUNK)
            f_src[b, pl.ds(j * 16, 16)] = sv
            f_dst[b, pl.ds(j * 16, 16)] = dv
        h = FIRE // 2
        c0 = pltpu.async_copy(m_hbm.at[f_src.at[b, pl.ds(0, h)]],
                              rows_v.at[b, pl.ds(0, h)], sem_g)
        c1 = pltpu.async_copy(m_hbm.at[f_src.at[b, pl.ds(h, h)]],
                              rows_v.at[b, pl.ds(h, h)], sem_g)
        c0.wait()
        c1.wait()
        pltpu.async_copy(rows_v.at[b], acc.at[f_dst.at[b]], sem_s, add=True)

    def chunk_body(p, _):
        lo = (cid * PASSES + p) * CHUNK

        def zrow(i, carry):
            for j in range(DP // 16):
                rows_v[0, i, pl.ds(j * 16, 16)] = jnp.zeros((16,), _f32)
            return carry
        lax.fori_loop(0, FIRE, zrow, 0)

        def zspan(i, carry):
            pltpu.sync_copy(rows_v.at[0, pl.ds(0, ZPT)],
                            acc.at[pl.ds(sid * (5 * ZPT) + i * ZPT, ZPT)])
            return carry
        lax.fori_loop(0, 5, zspan, 0)
        plsc.subcore_barrier()

        for j in range(FIRE // 16):
            f_dst[0, pl.ds(j * 16, 16)] = jnp.full((16,), CHUNK, jnp.int32)
            f_dst[1, pl.ds(j * 16, 16)] = jnp.full((16,), CHUNK, jnp.int32)
        pltpu.async_copy(rows_v.at[0], acc.at[f_dst.at[0]], sem_s, add=True)
        pltpu.async_copy(rows_v.at[1], acc.at[f_dst.at[1]], sem_s, add=True)
        row0 = sid * TROWS
        pltpu.async_copy(src_hbm.at[pl.ds(row0, BLKR)],
                         bsrc.at[pl.ds(0, BLKR)], sem_i)
        pltpu.async_copy(dst_hbm.at[pl.ds(row0, BLKR)],
                         bdst.at[pl.ds(0, BLKR)], sem_i)

        def blk_body(blk, carry):
            boff = pl.multiple_of(lax.rem(blk, 2) * BLKR, 8)
            pltpu.make_async_copy(src_hbm.at[pl.ds(0, BLKR)],
                                  bsrc.at[pl.ds(0, BLKR)], sem_i).wait()
            pltpu.make_async_copy(dst_hbm.at[pl.ds(0, BLKR)],
                                  bdst.at[pl.ds(0, BLKR)], sem_i).wait()

            def prefetch(_):
                nxt = sid * TROWS + (blk + 1) * BLKR
                noff = pl.multiple_of(lax.rem(blk + 1, 2) * BLKR, 8)
                pltpu.async_copy(src_hbm.at[pl.ds(nxt, BLKR)],
                                 bsrc.at[pl.ds(noff, BLKR)], sem_i)
                pltpu.async_copy(dst_hbm.at[pl.ds(nxt, BLKR)],
                                 bdst.at[pl.ds(noff, BLKR)], sem_i)
                return 0

            lax.cond(blk + 1 < NBLKS, prefetch, lambda _: 0, 0)

            def row_body(r, rc):
                cnt, k = rc
                for j in range(8):
                    s16 = bsrc[boff + r, pl.ds(j * 16, 16)]
                    d16 = bdst[boff + r, pl.ds(j * 16, 16)]
                    dl = d16 - lo
                    ok = (dl >= 0) & (dl < CHUNK)
                    plsc.store_compressed(st_src.at[pl.ds(cnt, 16)], s16,
                                          mask=ok)
                    plsc.store_compressed(st_dst.at[pl.ds(cnt, 16)], dl,
                                          mask=ok)
                    cnt = cnt + jnp.sum(jnp.where(ok, 1, 0))

                def do_fire(args):
                    cnt, k = args

                    def f0(_):
                        fire(0, None)
                        return 0

                    def f1(_):
                        fire(1, None)
                        return 0

                    lax.cond(lax.rem(k, 2) == 0, f0, f1, 0)
                    for j in range(8):
                        v = st_src[pl.ds(FIRE + j * 16, 16)]
                        st_src[pl.ds(j * 16, 16)] = v
                        v = st_dst[pl.ds(FIRE + j * 16, 16)]
                        st_dst[pl.ds(j * 16, 16)] = v
                    return cnt - FIRE, k + 1

                return lax.cond(cnt >= FIRE, do_fire, lambda a: a, (cnt, k))

            return lax.fori_loop(0, BLKR, row_body, carry)

        cnt, k = lax.fori_loop(0, NBLKS, blk_body, (jnp.int32(0), jnp.int32(0)))

        def g0(c):
            fire(0, c)
            return 0

        def g1(c):
            fire(1, c)
            return 0

        lax.cond(lax.rem(k, 2) == 0, g0, g1, cnt)
        scatter_wait()
        scatter_wait()
        plsc.subcore_barrier()

        def cspan(i, carry):
            start = sid * (7 * CPT) + i * CPT
            pltpu.sync_copy(acc.at[pl.ds(start, CPT)],
                            out_hbm.at[pl.ds(lo + start, CPT)])
            return carry
        lax.fori_loop(0, 7, cspan, 0)
        plsc.subcore_barrier()
        return _

    lax.fori_loop(0, PASSES, chunk_body, 0)


def _pad_w(w):
    return jnp.pad(w, ((0, DP - w.shape[0]), (0, DP - w.shape[1])))


def _pad_b(b):
    return jnp.pad(b, (0, DP - b.shape[0])).reshape(1, DP)


def kernel(nodes_feat, edge_index, edges_feat, nodes_num_norm_sqrt,
           edges_num_norm_sqrt, W_emb, b_emb, W_pool0, b_pool0, W_apply0,
           b_apply0, W_pool1, b_pool1, W_apply1, b_apply1):
    x = jnp.pad(nodes_feat, ((0, NP - N), (0, 0)))
    we = jnp.pad(W_emb, ((0, 0), (0, DP - H)))
    be = _pad_b(b_emb)
    wp0, bp0 = _pad_w(W_pool0), _pad_b(b_pool0)
    wp1, bp1 = _pad_w(W_pool1), _pad_b(b_pool1)
    wah0, wac0 = _pad_w(W_apply0[:H]), _pad_w(W_apply0[H:])
    ba0 = _pad_b(b_apply0)
    wah1, wac1 = _pad_w(W_apply1[:H]), _pad_w(W_apply1[H:])
    ba1 = _pad_b(b_apply1)

    src = jnp.pad(edge_index[0], (0, EP - E)).reshape(EROWS, 128)
    dst = jnp.pad(edge_index[1], (0, EP - E),
                  constant_values=jnp.int32(1 << 30)).reshape(EROWS, 128)

    h, m0 = _embed_pool(x, we, be, wp0, bp0)
    sums0 = _edge_pass(src, dst, m0)
    h1, m1 = _apply_pool(h, sums0, wah0, wac0, ba0, wp1, bp1)
    sums1 = _edge_pass(src, dst, m1)
    out = _final(h1, sums1, wah1, wac1, ba1)
    return out[:, :OUT_DIM]


# speedup vs baseline: 1.0015x; 1.0015x over previous
import functools

import jax
import jax.numpy as jnp
from jax import lax
from jax.experimental import pallas as pl
from jax.experimental.pallas import tpu as pltpu
from jax.experimental.pallas import tpu_sc as plsc

N = 50000
E = 800000
IN_DIM = 128
H = 108
OUT_DIM = 108

DP = 128
ONES_COL = 108
BR = 1024
NP = 50176
NBLK = NP // BR

NUM_CHUNKS = 8
CHUNK = NP // NUM_CHUNKS
PASSES = NUM_CHUNKS // 2
ACC_ROWS = CHUNK + 128
ZPT = 80
CPT = 56

FIRE = 128
STCAP = 272
EROWS = 6272
EP = EROWS * 128
TROWS = EROWS // 16
BLKR = 56
NBLKS = TROWS // BLKR

_f32 = jnp.float32


def _mm(a, b):
    return lax.dot(a, b, precision=lax.Precision.HIGHEST,
                   preferred_element_type=_f32)


def _ones_col(m):
    col = lax.broadcasted_iota(jnp.int32, m.shape, 1)
    return jnp.where(col == ONES_COL, 1.0, m)


def _embed_pool_body(x_ref, we_ref, be_ref, wp_ref, bp_ref, h_ref, m_ref):
    h = _mm(x_ref[...], we_ref[...]) + be_ref[...]
    h_ref[...] = h
    m = jnp.maximum(_mm(h, wp_ref[...]) + bp_ref[...], 0.0)
    m_ref[...] = _ones_col(m)


def _apply_block(h, s, wah, wac, ba):
    deg = s[:, ONES_COL:ONES_COL + 1]
    c = s * (1.0 / jnp.maximum(deg, 1.0))
    bundle = _mm(h, wah) + _mm(c, wac) + ba
    nrm = jnp.sqrt(jnp.sum(bundle * bundle, axis=1, keepdims=True))
    bundle = bundle / jnp.maximum(nrm, 1e-12)
    return h + jnp.maximum(bundle, 0.0)


def _apply_pool_body(h_ref, s_ref, wah_ref, wac_ref, ba_ref, wp_ref, bp_ref,
                     h1_ref, m1_ref):
    h1 = _apply_block(h_ref[...], s_ref[...], wah_ref[...], wac_ref[...],
                      ba_ref[...])
    h1_ref[...] = h1
    m = jnp.maximum(_mm(h1, wp_ref[...]) + bp_ref[...], 0.0)
    m1_ref[...] = _ones_col(m)


def _final_body(h_ref, s_ref, wah_ref, wac_ref, ba_ref, out_ref):
    pi = pl.program_id(0)
    h2 = _apply_block(h_ref[...], s_ref[...], wah_ref[...], wac_ref[...],
                      ba_ref[...])
    row = pi * BR + lax.broadcasted_iota(jnp.int32, h2.shape, 0)
    h2 = jnp.where(row < N, h2, 0.0)
    part = jnp.sum(h2, axis=0, keepdims=True)

    @pl.when(pi == 0)
    def _():
        out_ref[...] = jnp.zeros_like(out_ref)

    acc = out_ref[...] + part

    @pl.when(pi == NBLK - 1)
    def _():
        out_ref[...] = acc * (1.0 / N)

    @pl.when(pi < NBLK - 1)
    def _():
        out_ref[...] = acc


def _row_spec(i):
    return (i, 0)


def _fixed_spec(i):
    return (0, 0)


def _embed_pool(x, we, be, wp, bp):
    return pl.pallas_call(
        _embed_pool_body,
        grid=(NBLK,),
        in_specs=[
            pl.BlockSpec((BR, IN_DIM), _row_spec),
            pl.BlockSpec((IN_DIM, DP), _fixed_spec),
            pl.BlockSpec((1, DP), _fixed_spec),
            pl.BlockSpec((DP, DP), _fixed_spec),
            pl.BlockSpec((1, DP), _fixed_spec),
        ],
        out_specs=[pl.BlockSpec((BR, DP), _row_spec),
                   pl.BlockSpec((BR, DP), _row_spec)],
        out_shape=[jax.ShapeDtypeStruct((NP, DP), _f32)] * 2,
    )(x, we, be, wp, bp)


def _apply_pool(h, s, wah, wac, ba, wp, bp):
    return pl.pallas_call(
        _apply_pool_body,
        grid=(NBLK,),
        in_specs=[
            pl.BlockSpec((BR, DP), _row_spec),
            pl.BlockSpec((BR, DP), _row_spec),
            pl.BlockSpec((DP, DP), _fixed_spec),
            pl.BlockSpec((DP, DP), _fixed_spec),
            pl.BlockSpec((1, DP), _fixed_spec),
            pl.BlockSpec((DP, DP), _fixed_spec),
            pl.BlockSpec((1, DP), _fixed_spec),
        ],
        out_specs=[pl.BlockSpec((BR, DP), _row_spec),
                   pl.BlockSpec((BR, DP), _row_spec)],
        out_shape=[jax.ShapeDtypeStruct((NP, DP), _f32)] * 2,
    )(h, s, wah, wac, ba, wp, bp)


def _final(h, s, wah, wac, ba):
    return pl.pallas_call(
        _final_body,
        grid=(NBLK,),
        in_specs=[
            pl.BlockSpec((BR, DP), _row_spec),
            pl.BlockSpec((BR, DP), _row_spec),
            pl.BlockSpec((DP, DP), _fixed_spec),
            pl.BlockSpec((DP, DP), _fixed_spec),
            pl.BlockSpec((1, DP), _fixed_spec),
        ],
        out_specs=pl.BlockSpec((1, DP), _fixed_spec),
        out_shape=jax.ShapeDtypeStruct((1, DP), _f32),
    )(h, s, wah, wac, ba)


_SC_MESH = plsc.VectorSubcoreMesh(core_axis_name="c", subcore_axis_name="s")


@functools.partial(
    pl.kernel,
    out_type=jax.ShapeDtypeStruct((NP, DP), _f32),
    mesh=_SC_MESH,
    scratch_types=[
        pltpu.VMEM_SHARED((ACC_ROWS, DP), _f32),
        pltpu.VMEM((2, FIRE, DP), _f32),
        pltpu.VMEM((2 * BLKR, 128), jnp.int32),
        pltpu.VMEM((2 * BLKR, 128), jnp.int32),
        pltpu.VMEM((STCAP,), jnp.int32),
        pltpu.VMEM((STCAP,), jnp.int32),
        pltpu.VMEM((2, FIRE), jnp.int32),
        pltpu.VMEM((2, FIRE), jnp.int32),
        pltpu.SemaphoreType.DMA,
        pltpu.SemaphoreType.DMA,
        pltpu.SemaphoreType.DMA,
    ],
    compiler_params=pltpu.CompilerParams(needs_layout_passes=False),
)
def _edge_pass(src_hbm, dst_hbm, m_hbm, out_hbm,
               acc, rows_v, bsrc, bdst, st_src, st_dst, f_src, f_dst,
               sem_s, sem_g, sem_i):
    cid = lax.axis_index("c")
    sid = lax.axis_index("s")

    def scatter_wait():
        pltpu.make_async_copy(rows_v.at[0], acc.at[f_dst.at[0]], sem_s).wait()

    def gather_wait():
        pltpu.make_async_copy(m_hbm.at[f_src.at[0]], rows_v.at[0],
                              sem_g).wait()

    def fire(b, pad_cnt):
        scatter_wait()
        for j in range(FIRE // 16):
            sv = st_src[pl.ds(j * 16, 16)]
            dv = st_dst[pl.ds(j * 16, 16)]
            if pad_cnt is not None:
                i16 = lax.broadcasted_iota(jnp.int32, (16,), 0)
                keep = (j * 16 + i16) < pad_cnt
                sv = jnp.where(keep, sv, 0)
                dv = jnp.where(keep, dv, CHUNK)
            f_src[b, pl.ds(j * 16, 16)] = sv
            f_dst[b, pl.ds(j * 16, 16)] = dv
        h = FIRE // 2
        c0 = pltpu.async_copy(m_hbm.at[f_src.at[b, pl.ds(0, h)]],
                              rows_v.at[b, pl.ds(0, h)], sem_g)
        c1 = pltpu.async_copy(m_hbm.at[f_src.at[b, pl.ds(h, h)]],
                              rows_v.at[b, pl.ds(h, h)], sem_g)
        c0.wait()
        c1.wait()
        pltpu.async_copy(rows_v.at[b], acc.at[f_dst.at[b]], sem_s, add=True)

    def chunk_body(p, _):
        lo = (cid * PASSES + p) * CHUNK

        def zrow(i, carry):
            for j in range(DP // 16):
                rows_v[0, i, pl.ds(j * 16, 16)] = jnp.zeros((16,), _f32)
            return carry
        lax.fori_loop(0, FIRE, zrow, 0)

        def zspan(i, carry):
            pltpu.sync_copy(rows_v.at[0, pl.ds(0, ZPT)],
                            acc.at[pl.ds(sid * (5 * ZPT) + i * ZPT, ZPT)])
            return carry
        lax.fori_loop(0, 5, zspan, 0)
        plsc.subcore_barrier()

        for j in range(FIRE // 16):
            f_dst[0, pl.ds(j * 16, 16)] = jnp.full((16,), CHUNK, jnp.int32)
            f_dst[1, pl.ds(j * 16, 16)] = jnp.full((16,), CHUNK, jnp.int32)
        pltpu.async_copy(rows_v.at[0], acc.at[f_dst.at[0]], sem_s, add=True)
        pltpu.async_copy(rows_v.at[1], acc.at[f_dst.at[1]], sem_s, add=True)
        row0 = sid * TROWS
        pltpu.async_copy(src_hbm.at[pl.ds(row0, BLKR)],
                         bsrc.at[pl.ds(0, BLKR)], sem_i)
        pltpu.async_copy(dst_hbm.at[pl.ds(row0, BLKR)],
                         bdst.at[pl.ds(0, BLKR)], sem_i)

        def blk_body(blk, carry):
            boff = pl.multiple_of(lax.rem(blk, 2) * BLKR, 8)
            pltpu.make_async_copy(src_hbm.at[pl.ds(0, BLKR)],
                                  bsrc.at[pl.ds(0, BLKR)], sem_i).wait()
            pltpu.make_async_copy(dst_hbm.at[pl.ds(0, BLKR)],
                                  bdst.at[pl.ds(0, BLKR)], sem_i).wait()

            def prefetch(_):
                nxt = sid * TROWS + (blk + 1) * BLKR
                noff = pl.multiple_of(lax.rem(blk + 1, 2) * BLKR, 8)
                pltpu.async_copy(src_hbm.at[pl.ds(nxt, BLKR)],
                                 bsrc.at[pl.ds(noff, BLKR)], sem_i)
                pltpu.async_copy(dst_hbm.at[pl.ds(nxt, BLKR)],
                                 bdst.at[pl.ds(noff, BLKR)], sem_i)
                return 0

            lax.cond(blk + 1 < NBLKS, prefetch, lambda _: 0, 0)

            def row_body(r, rc):
                cnt, k = rc
                for j in range(8):
                    s16 = bsrc[boff + r, pl.ds(j * 16, 16)]
                    d16 = bdst[boff + r, pl.ds(j * 16, 16)]
                    dl = d16 - lo
                    ok = dl.astype(jnp.uint32) < jnp.uint32(CHUNK)
                    plsc.store_compressed(st_src.at[pl.ds(cnt, 16)], s16,
                                          mask=ok)
                    plsc.store_compressed(st_dst.at[pl.ds(cnt, 16)], dl,
                                          mask=ok)
                    cnt = cnt + jnp.sum(jnp.where(ok, 1, 0))

                def do_fire(args):
                    cnt, k = args

                    def f0(_):
                        fire(0, None)
                        return 0

                    def f1(_):
                        fire(1, None)
                        return 0

                    lax.cond(lax.rem(k, 2) == 0, f0, f1, 0)
                    for j in range(8):
                        v = st_src[pl.ds(FIRE + j * 16, 16)]
                        st_src[pl.ds(j * 16, 16)] = v
                        v = st_dst[pl.ds(FIRE + j * 16, 16)]
                        st_dst[pl.ds(j * 16, 16)] = v
                    return cnt - FIRE, k + 1

                return lax.cond(cnt >= FIRE, do_fire, lambda a: a, (cnt, k))

            return lax.fori_loop(0, BLKR, row_body, carry)

        cnt, k = lax.fori_loop(0, NBLKS, blk_body, (jnp.int32(0), jnp.int32(0)))

        def g0(c):
            fire(0, c)
            return 0

        def g1(c):
            fire(1, c)
            return 0

        lax.cond(lax.rem(k, 2) == 0, g0, g1, cnt)
        scatter_wait()
        scatter_wait()
        plsc.subcore_barrier()

        def cspan(i, carry):
            start = sid * (7 * CPT) + i * CPT
            pltpu.sync_copy(acc.at[pl.ds(start, CPT)],
                            out_hbm.at[pl.ds(lo + start, CPT)])
            return carry
        lax.fori_loop(0, 7, cspan, 0)
        plsc.subcore_barrier()
        return _

    lax.fori_loop(0, PASSES, chunk_body, 0)


def _pad_w(w):
    return jnp.pad(w, ((0, DP - w.shape[0]), (0, DP - w.shape[1])))


def _pad_b(b):
    return jnp.pad(b, (0, DP - b.shape[0])).reshape(1, DP)


def kernel(nodes_feat, edge_index, edges_feat, nodes_num_norm_sqrt,
           edges_num_norm_sqrt, W_emb, b_emb, W_pool0, b_pool0, W_apply0,
           b_apply0, W_pool1, b_pool1, W_apply1, b_apply1):
    x = jnp.pad(nodes_feat, ((0, NP - N), (0, 0)))
    we = jnp.pad(W_emb, ((0, 0), (0, DP - H)))
    be = _pad_b(b_emb)
    wp0, bp0 = _pad_w(W_pool0), _pad_b(b_pool0)
    wp1, bp1 = _pad_w(W_pool1), _pad_b(b_pool1)
    wah0, wac0 = _pad_w(W_apply0[:H]), _pad_w(W_apply0[H:])
    ba0 = _pad_b(b_apply0)
    wah1, wac1 = _pad_w(W_apply1[:H]), _pad_w(W_apply1[H:])
    ba1 = _pad_b(b_apply1)

    src = jnp.pad(edge_index[0], (0, EP - E)).reshape(EROWS, 128)
    dst = jnp.pad(edge_index[1], (0, EP - E),
                  constant_values=jnp.int32(1 << 30)).reshape(EROWS, 128)

    h, m0 = _embed_pool(x, we, be, wp0, bp0)
    sums0 = _edge_pass(src, dst, m0)
    h1, m1 = _apply_pool(h, sums0, wah0, wac0, ba0, wp1, bp1)
    sums1 = _edge_pass(src, dst, m1)
    out = _final(h1, sums1, wah1, wac1, ba1)
    return out[:, :OUT_DIM]
